# baseline (speedup 1.0000x reference)
import jax
import jax.numpy as jnp
from jax import lax
from jax.experimental import pallas as pl
from jax.experimental.pallas import tpu as pltpu
from jax.experimental.pallas import tpu_sc as plsc

_ALPHA = 0.1
_N = 32768
_D = 128
_K = 256
_NC = 2
_NS = 16
_NW = _NC * _NS
_TPW = _N // _NW
_CHUNK = 128
_NCHUNK = _TPW // _CHUNK
_COLS = 384
_ACC = _K * _COLS


def _vec_sqrt(x):
    i = lax.bitcast_convert_type(x, jnp.int32)
    i = jnp.int32(0x5F3759DF) - lax.shift_right_logical(i, 1)
    y = lax.bitcast_convert_type(i, jnp.float32)
    half = x * 0.5
    for _ in range(3):
        y = y * (1.5 - half * y * y)
    return jnp.where(x == 0.0, 0.0, x * y)


def _sc_body(emb, labels, mass, zinit, out, lab_v, mass_v, ebuf, acc):
    cid = lax.axis_index("c")
    sid = lax.axis_index("s")
    wid = sid * _NC + cid
    tok0 = wid * _TPW

    pltpu.sync_copy(zinit, acc)
    pltpu.sync_copy(labels.at[pl.ds(tok0, _TPW)], lab_v)
    pltpu.sync_copy(mass.at[pl.ds(tok0, _TPW)], mass_v)

    iota16 = lax.iota(jnp.int32, 16)

    def _prep(i, carry):
        sl = pl.ds(i * 16, 16)
        mass_v[sl] = _vec_sqrt(mass_v[sl])
        return carry
    lax.fori_loop(0, _TPW // 16, _prep, 0)

    cme = [iota16 + (j * 16) for j in range(_D // 16)]
    csq = iota16 + 272
    cmisc = iota16 + 288
    lane_lt8 = iota16 < 8

    def _one_tok(t, labsplat, msplat):
        sq = jnp.zeros((16,), jnp.float32)
        for j in range(_D // 16):
            ej = ebuf[t, pl.ds(j * 16, 16)]
            plsc.addupdate_scatter(acc, [labsplat, cme[j]], ej * msplat)
            plsc.addupdate_scatter(acc, [labsplat, cme[j] + _D], ej)
            sq = sq + ej * ej
        plsc.addupdate_scatter(acc, [labsplat, csq], sq)
        misc = jnp.where(lane_lt8, msplat, 1.0)
        plsc.addupdate_scatter(acc, [labsplat, cmisc], misc)

    def _chunk(ci, carry):
        pltpu.sync_copy(emb.at[pl.ds(tok0 + ci * _CHUNK, _CHUNK), :], ebuf)
        cb = ci * _CHUNK

        def _batch(b, c2):
            labv = []
            mv = []
            for s in range(2):
                bs = cb + 64 * s + b * 16
                labv.append(lab_v[pl.ds(bs, 16)])
                mv.append(mass_v[pl.ds(bs, 16)])
            for u in range(16):
                for s in range(2):
                    labsplat = jnp.full((16,), labv[s][u], jnp.int32)
                    msplat = jnp.full((16,), mv[s][u], jnp.float32)
                    _one_tok(64 * s + b * 16 + u, labsplat, msplat)
            return c2

        lax.fori_loop(0, 4, _batch, 0)
        return carry

    lax.fori_loop(0, _NCHUNK, _chunk, 0)

    pltpu.sync_copy(acc, out.at[wid])


_sc_stage1 = pl.kernel(
    _sc_body,
    out_type=jax.ShapeDtypeStruct((_NW, _K, _COLS), jnp.float32),
    mesh=plsc.VectorSubcoreMesh(
        core_axis_name="c", subcore_axis_name="s",
        num_cores=_NC, num_subcores=_NS),
    compiler_params=pltpu.CompilerParams(needs_layout_passes=False),
    scratch_types=[
        pltpu.VMEM((_TPW,), jnp.int32),
        pltpu.VMEM((_TPW,), jnp.float32),
        pltpu.VMEM((_CHUNK, _D), jnp.float32),
        pltpu.VMEM((_K, _COLS), jnp.float32),
    ],
)


def _tc_body(stats_ref, size_ref, o_ref):
    s = jnp.sum(stats_ref[...], axis=0)
    sme = s[:, 0:_D]
    s1 = s[:, _D:2 * _D]
    s2 = jnp.sum(s[:, 272:288], axis=1, keepdims=True)
    sm = s[:, 288:289]
    cnt = s[:, 296:297]
    c = sme / sm
    cs1 = jnp.sum(c * s1, axis=1, keepdims=True)
    cck = jnp.sum(c * c, axis=1, keepdims=True)
    intra = (s2 - 2.0 * cs1 + cnt * cck) / cnt
    loss_intra = jnp.sum(intra) / _K

    g = lax.dot_general(c, c, (((1,), (1,)), ((), ())),
                        preferred_element_type=jnp.float32)
    ccv = jnp.sum(c * c, axis=1)
    d2 = ccv[:, None] + ccv[None, :] - 2.0 * g
    pd = jnp.sqrt(jnp.maximum(d2, 0.0))
    q = jnp.sqrt(size_ref[0, :])
    qq = q[:, None] * q[None, :]
    ii = lax.broadcasted_iota(jnp.int32, (_K, _K), 0)
    jj = lax.broadcasted_iota(jnp.int32, (_K, _K), 1)
    off = ii != jj
    inter = jnp.sum(jnp.where(off, qq, 0.0) / jnp.where(off, pd, 1.0))
    loss_inter = _ALPHA * inter / (_K * (_K - 1))

    row = lax.broadcasted_iota(jnp.int32, (8, 128), 0)
    lane = lax.broadcasted_iota(jnp.int32, (8, 128), 1)
    vals = jnp.where(lane == 0, loss_intra + loss_inter,
                     jnp.where(lane == 1, loss_intra,
                               jnp.where(lane == 2, loss_inter, 0.0)))
    o_ref[...] = jnp.where(row == 0, vals, 0.0)


_tc_stage2 = pl.pallas_call(
    _tc_body,
    out_shape=jax.ShapeDtypeStruct((8, 128), jnp.float32),
)


def kernel(embeddings, labels, mass, size_map):
    zinit = jnp.zeros((_K, _COLS), jnp.float32)
    stats = _sc_stage1(embeddings, labels, mass, zinit)
    o = _tc_stage2(stats, size_map.reshape(1, _K))
    return (o[0, 0], o[0, 1], o[0, 2])


# speedup vs baseline: 2.4642x; 2.4642x over previous
import jax
import jax.numpy as jnp
from jax import lax
from jax.experimental import pallas as pl
from jax.experimental.pallas import tpu as pltpu
from jax.experimental.pallas import tpu_sc as plsc

_ALPHA = 0.1
_N = 32768
_D = 128
_K = 256
_NC = 2
_NS = 16
_NW = _NC * _NS
_KPW = _K // _NW
_CHUNK = 128
_CHUNKB = _CHUNK + 8
_NG = _N // 16
_COLS = 384


def _vec_sqrt(x):
    i = lax.bitcast_convert_type(x, jnp.int32)
    i = jnp.int32(0x5F3759DF) - lax.shift_right_logical(i, 1)
    y = lax.bitcast_convert_type(i, jnp.float32)
    half = x * 0.5
    for _ in range(3):
        y = y * (1.5 - half * y * y)
    return jnp.where(x == 0.0, 0.0, x * y)


def _sc_body(emb, labels, mass, out, lab_all, mass_all, ebuf, acc):
    cid = lax.axis_index("c")
    sid = lax.axis_index("s")
    wid = sid * _NC + cid
    lab0 = pl.multiple_of(wid * _KPW, _KPW)

    pltpu.sync_copy(labels, lab_all)
    pltpu.sync_copy(mass, mass_all)

    iota16 = lax.iota(jnp.int32, 16)
    zeros16 = jnp.zeros((16,), jnp.float32)
    lane_lt8 = iota16 < 8

    for r in range(_KPW):
        for j in range(_COLS // 16):
            acc[r, pl.ds(j * 16, 16)] = zeros16

    def _lower_bound(target):
        def step(_, lohi):
            lo, hi = lohi
            mid = lax.div(lo + hi, 2)
            v = lab_all[pl.ds(mid * 16, 16)][0]
            pred = v < target
            return (jnp.where(pred, mid, lo), jnp.where(pred, hi, mid))
        lo, _ = lax.fori_loop(
            0, 11, step, (jnp.int32(0), jnp.int32(_NG)))
        labg = lab_all[pl.ds(lo * 16, 16)]
        p = plsc.all_reduce_population_count(
            labg < jnp.full((16,), target, jnp.int32))
        p0 = p[0] if getattr(p, "ndim", 0) else p
        return lo * 16 + p0

    bounds = [_lower_bound(lab0 + r) for r in range(_KPW + 1)]
    start = bounds[0]
    end = bounds[_KPW]

    def _prep(i, carry):
        sl = pl.ds(i * 16, 16)
        mass_all[sl] = _vec_sqrt(mass_all[sl])
        return carry
    lax.fori_loop(lax.div(start, 16), lax.div(end + 15, 16), _prep, 0)

    def _run(l_lo, l_hi, d):
        init = ([zeros16] * 8, [zeros16] * 8, [zeros16] * 8, zeros16, zeros16)

        def tok(l, st):
            rme, rs1, rsq, rm, rcnt = st
            g = jnp.full((16,), d + l, jnp.int32)
            msplat = plsc.load_gather(mass_all, [g])
            rme2, rs12, rsq2 = [], [], []
            for j in range(8):
                ej = ebuf[l, pl.ds(j * 16, 16)]
                rme2.append(rme[j] + ej * msplat)
                rs12.append(rs1[j] + ej)
                rsq2.append(rsq[j] + ej * ej)
            return (rme2, rs12, rsq2, rm + msplat, rcnt + 1.0)

        return lax.fori_loop(l_lo, l_hi, tok, init)

    nch = lax.div(end - start + (_CHUNK - 1), _CHUNK)

    def _chunk(c, carry):
        b = start + c * _CHUNK
        d = pl.multiple_of(
            jnp.minimum(lax.div(b, 8) * 8, _N - _CHUNKB), 8)
        pltpu.sync_copy(emb.at[pl.ds(d, _CHUNKB), :], ebuf)
        l_lo = b - d
        l_hi = jnp.minimum(b + _CHUNK, end) - d
        for r in range(_KPW):
            r_lo = jnp.maximum(l_lo, bounds[r] - d)
            r_hi = jnp.minimum(l_hi, bounds[r + 1] - d)
            rme, rs1, rsq, rm, rcnt = _run(r_lo, r_hi, d)
            for j in range(8):
                plsc.addupdate(acc.at[r, pl.ds(j * 16, 16)], rme[j])
                plsc.addupdate(acc.at[r, pl.ds(_D + j * 16, 16)], rs1[j])
            sq = rsq[0]
            for j in range(1, 8):
                sq = sq + rsq[j]
            plsc.addupdate(acc.at[r, pl.ds(272, 16)], sq)
            misc = jnp.where(lane_lt8, rm, rcnt)
            plsc.addupdate(acc.at[r, pl.ds(288, 16)], misc)
        return carry

    lax.fori_loop(0, nch, _chunk, 0)

    pltpu.sync_copy(acc, out.at[pl.ds(lab0, _KPW), :])


_sc_stage1 = pl.kernel(
    _sc_body,
    out_type=jax.ShapeDtypeStruct((_K, _COLS), jnp.float32),
    mesh=plsc.VectorSubcoreMesh(
        core_axis_name="c", subcore_axis_name="s",
        num_cores=_NC, num_subcores=_NS),
    compiler_params=pltpu.CompilerParams(needs_layout_passes=False),
    scratch_types=[
        pltpu.VMEM((_N,), jnp.int32),
        pltpu.VMEM((_N,), jnp.float32),
        pltpu.VMEM((_CHUNKB, _D), jnp.float32),
        pltpu.VMEM((_KPW, _COLS), jnp.float32),
    ],
)


def _tc_body(stats_ref, size_ref, o_ref):
    s = stats_ref[...]
    sme = s[:, 0:_D]
    s1 = s[:, _D:2 * _D]
    s2 = jnp.sum(s[:, 272:288], axis=1, keepdims=True)
    sm = s[:, 288:289]
    cnt = s[:, 296:297]
    c = sme / sm
    cs1 = jnp.sum(c * s1, axis=1, keepdims=True)
    cck = jnp.sum(c * c, axis=1, keepdims=True)
    intra = (s2 - 2.0 * cs1 + cnt * cck) / cnt
    loss_intra = jnp.sum(intra) / _K

    g = lax.dot_general(c, c, (((1,), (1,)), ((), ())),
                        preferred_element_type=jnp.float32)
    ccv = jnp.sum(c * c, axis=1)
    d2 = ccv[:, None] + ccv[None, :] - 2.0 * g
    pd = jnp.sqrt(jnp.maximum(d2, 0.0))
    q = jnp.sqrt(size_ref[0, :])
    qq = q[:, None] * q[None, :]
    ii = lax.broadcasted_iota(jnp.int32, (_K, _K), 0)
    jj = lax.broadcasted_iota(jnp.int32, (_K, _K), 1)
    off = ii != jj
    inter = jnp.sum(jnp.where(off, qq, 0.0) / jnp.where(off, pd, 1.0))
    loss_inter = _ALPHA * inter / (_K * (_K - 1))

    row = lax.broadcasted_iota(jnp.int32, (8, 128), 0)
    lane = lax.broadcasted_iota(jnp.int32, (8, 128), 1)
    vals = jnp.where(lane == 0, loss_intra + loss_inter,
                     jnp.where(lane == 1, loss_intra,
                               jnp.where(lane == 2, loss_inter, 0.0)))
    o_ref[...] = jnp.where(row == 0, vals, 0.0)


_tc_stage2 = pl.pallas_call(
    _tc_body,
    out_shape=jax.ShapeDtypeStruct((8, 128), jnp.float32),
)


def kernel(embeddings, labels, mass, size_map):
    stats = _sc_stage1(embeddings, labels, mass)
    o = _tc_stage2(stats, size_map.reshape(1, _K))
    return (o[0, 0], o[0, 1], o[0, 2])


# speedup vs baseline: 2.8775x; 1.1677x over previous
import jax
import jax.numpy as jnp
from jax import lax
from jax.experimental import pallas as pl
from jax.experimental.pallas import tpu as pltpu
from jax.experimental.pallas import tpu_sc as plsc

_ALPHA = 0.1
_N = 32768
_D = 128
_K = 256
_NC = 2
_NS = 16
_NW = _NC * _NS
_KPW = _K // _NW
_CHUNK = 128
_CHUNKB = _CHUNK + 8
_NG = _N // 16
_COLS = 384


def _vec_sqrt(x):
    i = lax.bitcast_convert_type(x, jnp.int32)
    i = jnp.int32(0x5F3759DF) - lax.shift_right_logical(i, 1)
    y = lax.bitcast_convert_type(i, jnp.float32)
    half = x * 0.5
    for _ in range(3):
        y = y * (1.5 - half * y * y)
    return jnp.where(x == 0.0, 0.0, x * y)


def _sc_body(emb, labels, mass, out, lab_all, mass_all, ebuf, acc, sem):
    cid = lax.axis_index("c")
    sid = lax.axis_index("s")
    wid = sid * _NC + cid
    lab0 = pl.multiple_of(wid * _KPW, _KPW)

    pltpu.sync_copy(labels, lab_all)
    pltpu.sync_copy(mass, mass_all)

    iota16 = lax.iota(jnp.int32, 16)
    zeros16 = jnp.zeros((16,), jnp.float32)
    lane_lt8 = iota16 < 8

    for r in range(_KPW):
        for j in range(_COLS // 16):
            acc[r, pl.ds(j * 16, 16)] = zeros16

    def _lower_bound(target):
        def step(_, lohi):
            lo, hi = lohi
            mid = lax.div(lo + hi, 2)
            v = lab_all[pl.ds(mid * 16, 16)][0]
            pred = v < target
            return (jnp.where(pred, mid, lo), jnp.where(pred, hi, mid))
        lo, _ = lax.fori_loop(
            0, 11, step, (jnp.int32(0), jnp.int32(_NG)))
        labg = lab_all[pl.ds(lo * 16, 16)]
        p = plsc.all_reduce_population_count(
            labg < jnp.full((16,), target, jnp.int32))
        p0 = p[0] if getattr(p, "ndim", 0) else p
        return lo * 16 + p0

    bounds = [_lower_bound(lab0 + r) for r in range(_KPW + 1)]
    start = bounds[0]
    end = bounds[_KPW]

    def _prep(i, carry):
        sl = pl.ds(i * 16, 16)
        mass_all[sl] = _vec_sqrt(mass_all[sl])
        return carry
    lax.fori_loop(lax.div(start, 16), lax.div(end + 15, 16), _prep, 0)

    def _tok_body(buf, l, d, st):
        rme, rs1, rsq, rm = st
        g = jnp.full((16,), d + l, jnp.int32)
        msplat = plsc.load_gather(mass_all, [g])
        rme2, rs12, rsq2 = [], [], []
        for j in range(8):
            ej = ebuf[buf, l, pl.ds(j * 16, 16)]
            rme2.append(rme[j] + ej * msplat)
            rs12.append(rs1[j] + ej)
            rsq2.append(rsq[j] + ej * ej)
        return (rme2, rs12, rsq2, rm + msplat)

    def _run(buf, l_lo, l_hi, d):
        init = ([zeros16] * 8, [zeros16] * 8, [zeros16] * 8, zeros16)
        count = jnp.maximum(l_hi - l_lo, 0)
        half = lax.div(count, 2)

        def tok2(i, st):
            l = l_lo + i * 2
            return _tok_body(buf, l + 1, d, _tok_body(buf, l, d, st))

        st = lax.fori_loop(0, half, tok2, init)
        st = lax.cond(count == half * 2,
                      lambda s: s,
                      lambda s: _tok_body(buf, l_lo + count - 1, d, s),
                      st)
        return st, count

    nch = lax.div(end - start + (_CHUNK - 1), _CHUNK)

    def _dma_base(c):
        b = start + c * _CHUNK
        d = pl.multiple_of(
            jnp.minimum(lax.div(b, 8) * 8, _N - _CHUNKB), 8)
        return b, d

    def _dma_start(c):
        _, d = _dma_base(c)
        pltpu.async_copy(
            emb.at[pl.ds(d, _CHUNKB), :], ebuf.at[jnp.bitwise_and(c, 1)],
            sem)

    @pl.when(nch > 0)
    def _():
        _dma_start(0)

    def _chunk(c, carry):
        @pl.when(c + 1 < nch)
        def _():
            _dma_start(c + 1)
        b, d = _dma_base(c)
        buf = jnp.bitwise_and(c, 1)
        pltpu.make_async_copy(
            emb.at[pl.ds(d, _CHUNKB), :], ebuf.at[buf], sem).wait()
        l_lo = b - d
        l_hi = jnp.minimum(b + _CHUNK, end) - d
        for r in range(_KPW):
            r_lo = jnp.maximum(l_lo, bounds[r] - d)
            r_hi = jnp.minimum(l_hi, bounds[r + 1] - d)
            (rme, rs1, rsq, rm), count = _run(buf, r_lo, r_hi, d)
            for j in range(8):
                plsc.addupdate(acc.at[r, pl.ds(j * 16, 16)], rme[j])
                plsc.addupdate(acc.at[r, pl.ds(_D + j * 16, 16)], rs1[j])
            sq = rsq[0]
            for j in range(1, 8):
                sq = sq + rsq[j]
            plsc.addupdate(acc.at[r, pl.ds(272, 16)], sq)
            cntf = jnp.full((16,), count.astype(jnp.float32))
            misc = jnp.where(lane_lt8, rm, cntf)
            plsc.addupdate(acc.at[r, pl.ds(288, 16)], misc)
        return carry

    lax.fori_loop(0, nch, _chunk, 0)

    pltpu.sync_copy(acc, out.at[pl.ds(lab0, _KPW), :])


_sc_stage1 = pl.kernel(
    _sc_body,
    out_type=jax.ShapeDtypeStruct((_K, _COLS), jnp.float32),
    mesh=plsc.VectorSubcoreMesh(
        core_axis_name="c", subcore_axis_name="s",
        num_cores=_NC, num_subcores=_NS),
    compiler_params=pltpu.CompilerParams(needs_layout_passes=False),
    scratch_types=[
        pltpu.VMEM((_N,), jnp.int32),
        pltpu.VMEM((_N,), jnp.float32),
        pltpu.VMEM((2, _CHUNKB, _D), jnp.float32),
        pltpu.VMEM((_KPW, _COLS), jnp.float32),
        pltpu.SemaphoreType.DMA,
    ],
)


def _tc_body(stats_ref, size_ref, o_ref):
    s = stats_ref[...]
    sme = s[:, 0:_D]
    s1 = s[:, _D:2 * _D]
    s2 = jnp.sum(s[:, 272:288], axis=1, keepdims=True)
    sm = s[:, 288:289]
    cnt = s[:, 296:297]
    c = sme / sm
    cs1 = jnp.sum(c * s1, axis=1, keepdims=True)
    cck = jnp.sum(c * c, axis=1, keepdims=True)
    intra = (s2 - 2.0 * cs1 + cnt * cck) / cnt
    loss_intra = jnp.sum(intra) / _K

    g = lax.dot_general(c, c, (((1,), (1,)), ((), ())),
                        preferred_element_type=jnp.float32)
    ccv = jnp.sum(c * c, axis=1)
    d2 = ccv[:, None] + ccv[None, :] - 2.0 * g
    pd = jnp.sqrt(jnp.maximum(d2, 0.0))
    q = jnp.sqrt(size_ref[0, :])
    qq = q[:, None] * q[None, :]
    ii = lax.broadcasted_iota(jnp.int32, (_K, _K), 0)
    jj = lax.broadcasted_iota(jnp.int32, (_K, _K), 1)
    off = ii != jj
    inter = jnp.sum(jnp.where(off, qq, 0.0) / jnp.where(off, pd, 1.0))
    loss_inter = _ALPHA * inter / (_K * (_K - 1))

    row = lax.broadcasted_iota(jnp.int32, (8, 128), 0)
    lane = lax.broadcasted_iota(jnp.int32, (8, 128), 1)
    vals = jnp.where(lane == 0, loss_intra + loss_inter,
                     jnp.where(lane == 1, loss_intra,
                               jnp.where(lane == 2, loss_inter, 0.0)))
    o_ref[...] = jnp.where(row == 0, vals, 0.0)


_tc_stage2 = pl.pallas_call(
    _tc_body,
    out_shape=jax.ShapeDtypeStruct((8, 128), jnp.float32),
)


def kernel(embeddings, labels, mass, size_map):
    stats = _sc_stage1(embeddings, labels, mass)
    o = _tc_stage2(stats, size_map.reshape(1, _K))
    return (o[0, 0], o[0, 1], o[0, 2])
